# baseline (speedup 1.0000x reference)
import functools

import jax
import jax.numpy as jnp
from jax.experimental import pallas as pl

N = 4096
NC = 30
WIN = 100
D = 300
ATT_K = 25
MBLK = 8

NEG = -1e10


def _fused_body(emb_ref, cmask_ref, tok_ref, tmask_ref, b1_ref, b2_ref, out_ref):
    b2 = b2_ref[...]
    b1 = b1_ref[...]

    ts_rows = []
    for m in range(MBLK):
        embw = emb_ref[m] * b2
        s = jax.lax.dot_general(
            embw, tok_ref[m],
            dimension_numbers=(((1,), (1,)), ((), ())),
            preferred_element_type=jnp.float32)
        smax = jnp.max(s, axis=0, keepdims=True)
        ts_rows.append(smax)
    ts = jnp.concatenate(ts_rows, axis=0)
    ts = jnp.where(tmask_ref[...] > 0, ts, NEG)

    work = ts
    thr = jnp.max(work, axis=1, keepdims=True)
    m0 = thr
    for _ in range(ATT_K - 1):
        work = jnp.where(work >= thr, -jnp.inf, work)
        thr = jnp.max(work, axis=1, keepdims=True)

    ex = jnp.where(ts >= thr, jnp.exp(ts - m0), 0.0)
    probs = ex / jnp.sum(ex, axis=1, keepdims=True)
    probs_t = probs.T

    for m in range(MBLK):
        pm = probs_t[:, m:m + 1]
        fcs = jnp.sum(tok_ref[m] * pm, axis=0, keepdims=True)
        g = fcs * b1
        v = jnp.sum(emb_ref[m] * g, axis=1)
        out_ref[m, :] = jnp.where(cmask_ref[m, :] > 0, v, NEG)


@jax.jit
def _run(embeddings, cmask, token_embeddings, tmask, b1, b2):
    grid = (N // MBLK,)
    return pl.pallas_call(
        _fused_body,
        grid=grid,
        in_specs=[
            pl.BlockSpec((MBLK, NC, D), lambda i: (i, 0, 0)),
            pl.BlockSpec((MBLK, NC), lambda i: (i, 0)),
            pl.BlockSpec((MBLK, WIN, D), lambda i: (i, 0, 0)),
            pl.BlockSpec((MBLK, WIN), lambda i: (i, 0)),
            pl.BlockSpec((1, D), lambda i: (0, 0)),
            pl.BlockSpec((1, D), lambda i: (0, 0)),
        ],
        out_specs=pl.BlockSpec((MBLK, NC), lambda i: (i, 0)),
        out_shape=jax.ShapeDtypeStruct((N, NC), jnp.float32),
    )(embeddings, cmask, token_embeddings, tmask, b1, b2)


def kernel(n, embeddings, masks, token_embeddings, token_masks, B_diag1, B_diag2):
    del n
    cmask = masks.astype(jnp.int32)
    tmask = token_masks.astype(jnp.int32)
    b1 = B_diag1.reshape(1, D)
    b2 = B_diag2.reshape(1, D)
    return _run(embeddings, cmask, token_embeddings, tmask, b1, b2)


# speedup vs baseline: 1.5451x; 1.5451x over previous
import jax
import jax.numpy as jnp
from jax.experimental import pallas as pl

N = 4096
NC = 30
WIN = 100
D = 300
ATT_K = 25
MBLK = 64

NEG = -1e10


def _fused_body(emb_ref, cmask_ref, tok_ref, tmask_ref, b1_ref, b2_ref, out_ref):
    b2 = b2_ref[...]
    b1 = b1_ref[...]

    ts_rows = []
    for m in range(MBLK):
        embw = emb_ref[m] * b2
        s = jax.lax.dot_general(
            embw, tok_ref[m],
            dimension_numbers=(((1,), (1,)), ((), ())),
            preferred_element_type=jnp.float32)
        smax = jnp.max(s, axis=0, keepdims=True)
        ts_rows.append(smax)
    ts = jnp.concatenate(ts_rows, axis=0)
    ts = jnp.where(tmask_ref[...] > 0, ts, NEG)
    tst = ts.T

    work = tst
    thr = jnp.max(work, axis=0, keepdims=True)
    m0 = thr
    for _ in range(ATT_K - 1):
        work = jnp.where(work >= thr, -jnp.inf, work)
        thr = jnp.max(work, axis=0, keepdims=True)

    ex = jnp.where(tst >= thr, jnp.exp(tst - m0), 0.0)
    probs_t = ex / jnp.sum(ex, axis=0, keepdims=True)

    for m in range(MBLK):
        pm = probs_t[:, m:m + 1]
        fcs = jax.lax.dot_general(
            pm, tok_ref[m],
            dimension_numbers=(((0,), (0,)), ((), ())),
            preferred_element_type=jnp.float32)
        g = fcs * b1
        v = jax.lax.dot_general(
            g, emb_ref[m],
            dimension_numbers=(((1,), (1,)), ((), ())),
            preferred_element_type=jnp.float32)
        out_ref[m, :] = jnp.where(cmask_ref[m, :] > 0, v[0], NEG)


@jax.jit
def _run(embeddings, cmask, token_embeddings, tmask, b1, b2):
    grid = (N // MBLK,)
    return pl.pallas_call(
        _fused_body,
        grid=grid,
        in_specs=[
            pl.BlockSpec((MBLK, NC, D), lambda i: (i, 0, 0)),
            pl.BlockSpec((MBLK, NC), lambda i: (i, 0)),
            pl.BlockSpec((MBLK, WIN, D), lambda i: (i, 0, 0)),
            pl.BlockSpec((MBLK, WIN), lambda i: (i, 0)),
            pl.BlockSpec((1, D), lambda i: (0, 0)),
            pl.BlockSpec((1, D), lambda i: (0, 0)),
        ],
        out_specs=pl.BlockSpec((MBLK, NC), lambda i: (i, 0)),
        out_shape=jax.ShapeDtypeStruct((N, NC), jnp.float32),
    )(embeddings, cmask, token_embeddings, tmask, b1, b2)


def kernel(n, embeddings, masks, token_embeddings, token_masks, B_diag1, B_diag2):
    del n
    cmask = masks.astype(jnp.int32)
    tmask = token_masks.astype(jnp.int32)
    b1 = B_diag1.reshape(1, D)
    b2 = B_diag2.reshape(1, D)
    return _run(embeddings, cmask, token_embeddings, tmask, b1, b2)


# speedup vs baseline: 1.7899x; 1.1585x over previous
import jax
import jax.numpy as jnp
from jax.experimental import pallas as pl

N = 4096
NC = 30
WIN = 100
D = 300
ATT_K = 25
MBLK = 128

NEG = -1e10


def _fused_body(embc_ref, cmask_ref, tok_ref, tmask_ref, out_ref):
    ts_rows = []
    s2_list = []
    for m in range(MBLK):
        s = jax.lax.dot_general(
            embc_ref[m], tok_ref[m],
            dimension_numbers=(((1,), (1,)), ((), ())),
            preferred_element_type=jnp.float32)
        ts_rows.append(jnp.max(s[:NC], axis=0, keepdims=True))
        s2_list.append(s[NC:])
    ts = jnp.concatenate(ts_rows, axis=0)
    ts = jnp.where(tmask_ref[...] > 0, ts, NEG)
    tst = ts.T

    work = tst
    thr = jnp.max(work, axis=0, keepdims=True)
    m0 = thr
    for _ in range(ATT_K - 1):
        work = jnp.where(work >= thr, -jnp.inf, work)
        thr = jnp.max(work, axis=0, keepdims=True)

    ex = jnp.where(tst >= thr, jnp.exp(tst - m0), 0.0)
    probs_t = ex / jnp.sum(ex, axis=0, keepdims=True)
    probs = probs_t.T

    for m in range(MBLK):
        v = jax.lax.dot_general(
            probs[m:m + 1], s2_list[m],
            dimension_numbers=(((1,), (1,)), ((), ())),
            preferred_element_type=jnp.float32)
        out_ref[m, :] = jnp.where(cmask_ref[m, :] > 0, v[0], NEG)


@jax.jit
def _run(embc, cmask, tok, tmask):
    grid = (N // MBLK,)
    return pl.pallas_call(
        _fused_body,
        grid=grid,
        in_specs=[
            pl.BlockSpec((MBLK, 2 * NC, D), lambda i: (i, 0, 0)),
            pl.BlockSpec((MBLK, NC), lambda i: (i, 0)),
            pl.BlockSpec((MBLK, WIN, D), lambda i: (i, 0, 0)),
            pl.BlockSpec((MBLK, WIN), lambda i: (i, 0)),
        ],
        out_specs=pl.BlockSpec((MBLK, NC), lambda i: (i, 0)),
        out_shape=jax.ShapeDtypeStruct((N, NC), jnp.float32),
    )(embc, cmask, tok, tmask)


def kernel(n, embeddings, masks, token_embeddings, token_masks, B_diag1, B_diag2):
    del n
    embc = jnp.concatenate(
        [embeddings * B_diag2[None, None, :],
         embeddings * B_diag1[None, None, :]], axis=1).astype(jnp.bfloat16)
    tok = token_embeddings.astype(jnp.bfloat16)
    cmask = masks.astype(jnp.int32)
    tmask = token_masks.astype(jnp.int32)
    return _run(embc, cmask, tok, tmask)


# speedup vs baseline: 2.1941x; 1.2258x over previous
import jax
import jax.numpy as jnp
from jax.experimental import pallas as pl

N = 4096
NC = 30
WIN = 100
D = 300
ATT_K = 25
MBLK = 64

NEG = -1e10


def _fused_body(emb_ref, cmask_ref, tok_ref, tmask_ref, b1_ref, b2_ref, out_ref):
    b2 = b2_ref[...]
    b1 = b1_ref[...]

    ts_rows = []
    s2_list = []
    for m in range(MBLK):
        embc = jnp.concatenate([emb_ref[m] * b2, emb_ref[m] * b1], axis=0)
        s = jax.lax.dot_general(
            embc, tok_ref[m],
            dimension_numbers=(((1,), (1,)), ((), ())),
            preferred_element_type=jnp.float32)
        ts_rows.append(jnp.max(s[:NC], axis=0, keepdims=True))
        s2_list.append(s[NC:])
    ts = jnp.concatenate(ts_rows, axis=0)
    ts = jnp.where(tmask_ref[...] > 0, ts, NEG)
    tst = ts.T

    work = tst
    thr = jnp.max(work, axis=0, keepdims=True)
    m0 = thr
    for _ in range(ATT_K - 1):
        work = jnp.where(work >= thr, -jnp.inf, work)
        thr = jnp.max(work, axis=0, keepdims=True)

    ex = jnp.where(tst >= thr, jnp.exp(tst - m0), 0.0)
    probs_t = ex / jnp.sum(ex, axis=0, keepdims=True)
    probs = probs_t.T

    for m in range(MBLK):
        v = jax.lax.dot_general(
            probs[m:m + 1], s2_list[m],
            dimension_numbers=(((1,), (1,)), ((), ())),
            preferred_element_type=jnp.float32)
        out_ref[m, :] = jnp.where(cmask_ref[m, :] > 0, v[0], NEG)


@jax.jit
def _run(embeddings, cmask, token_embeddings, tmask, b1, b2):
    grid = (N // MBLK,)
    return pl.pallas_call(
        _fused_body,
        grid=grid,
        in_specs=[
            pl.BlockSpec((MBLK, NC, D), lambda i: (i, 0, 0)),
            pl.BlockSpec((MBLK, NC), lambda i: (i, 0)),
            pl.BlockSpec((MBLK, WIN, D), lambda i: (i, 0, 0)),
            pl.BlockSpec((MBLK, WIN), lambda i: (i, 0)),
            pl.BlockSpec((1, D), lambda i: (0, 0)),
            pl.BlockSpec((1, D), lambda i: (0, 0)),
        ],
        out_specs=pl.BlockSpec((MBLK, NC), lambda i: (i, 0)),
        out_shape=jax.ShapeDtypeStruct((N, NC), jnp.float32),
    )(embeddings, cmask, token_embeddings, tmask, b1, b2)


def kernel(n, embeddings, masks, token_embeddings, token_masks, B_diag1, B_diag2):
    del n
    cmask = masks.astype(jnp.int32)
    tmask = token_masks.astype(jnp.int32)
    b1 = B_diag1.reshape(1, D)
    b2 = B_diag2.reshape(1, D)
    return _run(embeddings, cmask, token_embeddings, tmask, b1, b2)
